# baseline (speedup 1.0000x reference)
import jax
import jax.numpy as jnp
from jax import lax
from jax.experimental import pallas as pl
from jax.experimental.pallas import tpu as pltpu, tpu_sc as plsc

_NC, _NS = 2, 16
_NW = _NC * _NS
_LANES = 16
_G = 40
_NBUF = 4


def _make_body(seq_len, d, chunks_per_w, gathers_per_chunk):
    rows_per_w = chunks_per_w * gathers_per_chunk

    def body(x_hbm, tok_hbm, pos_hbm, out_hbm, idx_v, pos_v, bufs, *sems):
        gsem = sems[:_NBUF]
        ssem = sems[_NBUF:]
        wid = lax.axis_index("s") * _NC + lax.axis_index("c")
        seq0 = wid * chunks_per_w

        pltpu.sync_copy(x_hbm.at[pl.ds(wid * rows_per_w, rows_per_w)], idx_v)
        pltpu.sync_copy(pos_hbm, pos_v)

        def fire_gather(g, b):
            for j in range(gathers_per_chunk):
                pltpu.async_copy(
                    tok_hbm.at[idx_v.at[g * gathers_per_chunk + j]],
                    bufs.at[b, pl.ds(j * _G, _G)],
                    gsem[b])

        def wait_gather(b):
            pltpu.make_async_copy(
                tok_hbm.at[pl.ds(0, seq_len)], bufs.at[b], gsem[b]).wait()

        def fire_store(g, b):
            pltpu.async_copy(
                bufs.at[b], out_hbm.at[pl.ds((seq0 + g) * seq_len, seq_len)],
                ssem[b])

        def wait_store(b):
            pltpu.make_async_copy(
                bufs.at[b], out_hbm.at[pl.ds(0, seq_len)], ssem[b]).wait()

        for b in range(_NBUF - 1):
            fire_gather(b, b)

        @pl.loop(0, chunks_per_w, step=_NBUF)
        def _(gbase):
            for b in range(_NBUF):
                g = gbase + b
                wait_gather(b)
                g2 = g + _NBUF - 1
                b2 = (b + _NBUF - 1) % _NBUF

                @pl.when(g2 < chunks_per_w)
                def _():
                    @pl.when(g2 >= _NBUF)
                    def _():
                        wait_store(b2)
                    fire_gather(g2, b2)

                @pl.loop(0, seq_len)
                def _(r):
                    for c in range(d // _LANES):
                        v = pos_v[r, pl.ds(c * _LANES, _LANES)]
                        plsc.addupdate(
                            bufs.at[b, r, pl.ds(c * _LANES, _LANES)], v)

                fire_store(g, b)

        for b in range(_NBUF):
            wait_store(b)

    return body


def kernel(x, token_emb, pos_emb):
    batch, seq_len = x.shape
    d = token_emb.shape[1]
    total = batch * seq_len
    assert seq_len % _G == 0 and batch % (_NW * _NBUF) == 0
    gathers_per_chunk = seq_len // _G
    chunks_per_w = batch // _NW

    x2d = x.reshape(total // _G, _G)
    body = _make_body(seq_len, d, chunks_per_w, gathers_per_chunk)
    rows_per_w = chunks_per_w * gathers_per_chunk

    out = pl.kernel(
        body,
        out_type=jax.ShapeDtypeStruct((total, d), jnp.float32),
        mesh=plsc.VectorSubcoreMesh(core_axis_name="c", subcore_axis_name="s"),
        scratch_types=[
            pltpu.VMEM((rows_per_w, _G), jnp.int32),
            pltpu.VMEM((seq_len, d), jnp.float32),
            pltpu.VMEM((_NBUF, seq_len, d), jnp.float32),
        ] + [pltpu.SemaphoreType.DMA] * (2 * _NBUF),
        compiler_params=pltpu.CompilerParams(use_tc_tiling_on_sc=False),
    )(x2d, token_emb, pos_emb)
    return out.reshape(batch, seq_len, d)
